# Initial kernel scaffold; baseline (speedup 1.0000x reference)
import functools

import jax
import jax.numpy as jnp
from jax import lax
from jax.experimental import pallas as pl
from jax.experimental.pallas import tpu as pltpu
from jax.experimental.pallas import tpu_sc as plsc

N_NODES = 10000
N_EDGES = 320000
D_NODE = 128
DE = 16

NC = 2
NS = 16
NW = NC * NS
EPW = N_EDGES // NW
GC = 80
CH = 2000
NGC = CH // GC
NCH = EPW // CH
IDX_ROWS = EPW // GC


def _proj_body(ns_ref, w_ref, ps_ref, pd_ref):
    p = jnp.dot(ns_ref[...], w_ref[...], preferred_element_type=jnp.float32)
    ps_ref[...] = p[:, 0:DE]
    pd_ref[...] = p[:, DE:2 * DE]


def _node_project(node_scalars, w_sd):
    return pl.pallas_call(
        _proj_body,
        out_shape=[
            jax.ShapeDtypeStruct((N_NODES, DE), jnp.float32),
            jax.ShapeDtypeStruct((N_NODES, DE), jnp.float32),
        ],
    )(node_scalars, w_sd)


def _gather_body(ps_hbm, pd_hbm, si_hbm, di_hbm, gs_hbm, gd_hbm,
                 si_v, di_v, rs_v, rd_v, sem_s, sem_d):
    wid = lax.axis_index("s") * NC + lax.axis_index("c")
    row0 = wid * IDX_ROWS
    base = wid * EPW
    pltpu.sync_copy(si_hbm.at[pl.ds(row0, IDX_ROWS)], si_v)
    pltpu.sync_copy(di_hbm.at[pl.ds(row0, IDX_ROWS)], di_v)
    for c in range(NCH):
        descs = []
        for j in range(NGC):
            r = c * NGC + j
            descs.append(pltpu.async_copy(
                ps_hbm.at[si_v.at[r]], rs_v.at[pl.ds(j * GC, GC)], sem_s))
            descs.append(pltpu.async_copy(
                pd_hbm.at[di_v.at[r]], rd_v.at[pl.ds(j * GC, GC)], sem_d))
        for dsc in descs:
            dsc.wait()
        pltpu.sync_copy(rs_v, gs_hbm.at[pl.ds(base + c * CH, CH)])
        pltpu.sync_copy(rd_v, gd_hbm.at[pl.ds(base + c * CH, CH)])


_sc_gather = functools.partial(
    pl.kernel,
    out_type=[
        jax.ShapeDtypeStruct((N_EDGES, DE), jnp.float32),
        jax.ShapeDtypeStruct((N_EDGES, DE), jnp.float32),
    ],
    mesh=plsc.VectorSubcoreMesh(core_axis_name="c", subcore_axis_name="s"),
    scratch_types=[
        pltpu.VMEM((IDX_ROWS, GC), jnp.int32),
        pltpu.VMEM((IDX_ROWS, GC), jnp.int32),
        pltpu.VMEM((CH, DE), jnp.float32),
        pltpu.VMEM((CH, DE), jnp.float32),
        pltpu.SemaphoreType.DMA,
        pltpu.SemaphoreType.DMA,
    ],
)(_gather_body)


ROWS = N_EDGES * DE // 128
BE = 4000
NB = ROWS // BE


def _epi_body(gs, gd, ef, dd, w1e, w1d, w2, mavg, b1t, b2t, gt, bt, out):
    x = gs[...] + gd[...] + b1t[...]
    x = x + jnp.dot(ef[...], w1e[...], preferred_element_type=jnp.float32)
    x = x + jnp.dot(dd[...], w1d[...], preferred_element_type=jnp.float32)
    h1 = x * jax.nn.sigmoid(x)
    y = jnp.dot(h1, w2[...], preferred_element_type=jnp.float32) + b2t[...]
    h2 = y * jax.nn.sigmoid(y)
    r = ef[...] + h2
    m = jnp.dot(r, mavg[...], preferred_element_type=jnp.float32)
    q = r - m
    v = jnp.dot(q * q, mavg[...], preferred_element_type=jnp.float32)
    out[...] = q * lax.rsqrt(v + 1e-5) * gt[...] + bt[...]


def _epilogue(gs_p, gd_p, ef_p, d_p, bd1e, bd1d, bd2, mavg, b1t, b2t, gt, bt):
    big = pl.BlockSpec((BE, 128), lambda i: (i, 0))
    wsp = pl.BlockSpec((128, 128), lambda i: (0, 0))
    vsp = pl.BlockSpec((1, 128), lambda i: (0, 0))
    return pl.pallas_call(
        _epi_body,
        grid=(NB,),
        in_specs=[big, big, big, big, wsp, wsp, wsp, wsp, vsp, vsp, vsp, vsp],
        out_specs=big,
        out_shape=jax.ShapeDtypeStruct((ROWS, 128), jnp.float32),
    )(gs_p, gd_p, ef_p, d_p, bd1e, bd1d, bd2, mavg, b1t, b2t, gt, bt)


def kernel(node_scalars, edge_feats, d, src_idxs, dst_idxs,
           W1, b1, W2, b2, ln_g, ln_b):
    w_sd = jnp.concatenate([W1[0:D_NODE], W1[D_NODE:2 * D_NODE]], axis=1)
    eye8 = jnp.eye(8, dtype=jnp.float32)
    bd1e = jnp.kron(eye8, W1[2 * D_NODE:2 * D_NODE + DE])
    bd1d = jnp.kron(eye8, W1[2 * D_NODE + DE:2 * D_NODE + 2 * DE])
    bd2 = jnp.kron(eye8, W2)
    mavg = jnp.kron(eye8, jnp.full((DE, DE), 1.0 / DE, dtype=jnp.float32))
    b1t = jnp.tile(b1, 8)[None, :]
    b2t = jnp.tile(b2, 8)[None, :]
    gt = jnp.tile(ln_g, 8)[None, :]
    bt = jnp.tile(ln_b, 8)[None, :]

    psrc, pdst = _node_project(node_scalars, w_sd)

    si2 = src_idxs.astype(jnp.int32).reshape(N_EDGES // GC, GC)
    di2 = dst_idxs.astype(jnp.int32).reshape(N_EDGES // GC, GC)
    gs, gd = _sc_gather(psrc, pdst, si2, di2)

    out_p = _epilogue(
        gs.reshape(ROWS, 128), gd.reshape(ROWS, 128),
        edge_feats.reshape(ROWS, 128), d.reshape(ROWS, 128),
        bd1e, bd1d, bd2, mavg, b1t, b2t, gt, bt)
    return out_p.reshape(N_EDGES, DE)


# speedup vs baseline: 4.2004x; 4.2004x over previous
import functools

import jax
import jax.numpy as jnp
from jax import lax
from jax.experimental import pallas as pl
from jax.experimental.pallas import tpu as pltpu
from jax.experimental.pallas import tpu_sc as plsc

N_NODES = 10000
N_EDGES = 320000
D_NODE = 128
DE = 16

NC = 2
NS = 16
NW = NC * NS
EPW = N_EDGES // NW
GC = 80
CH = 2000
NGC = CH // GC
NCH = EPW // CH
IDX_ROWS = EPW // GC


def _proj_body(ns_ref, w_ref, ps_ref, pd_ref):
    p = jnp.dot(ns_ref[...], w_ref[...], preferred_element_type=jnp.float32)
    ps_ref[...] = p[:, 0:DE]
    pd_ref[...] = p[:, DE:2 * DE]


def _node_project(node_scalars, w_sd):
    return pl.pallas_call(
        _proj_body,
        out_shape=[
            jax.ShapeDtypeStruct((N_NODES, DE), jnp.float32),
            jax.ShapeDtypeStruct((N_NODES, DE), jnp.float32),
        ],
    )(node_scalars, w_sd)


def _gather_body(ps_hbm, pd_hbm, si_hbm, di_hbm, gs_hbm, gd_hbm,
                 si_v, di_v, rs_v, rd_v, sem_s, sem_d):
    wid = lax.axis_index("s") * NC + lax.axis_index("c")
    base = wid * EPW
    pltpu.sync_copy(si_hbm.at[wid], si_v)
    pltpu.sync_copy(di_hbm.at[wid], di_v)
    for c in range(NCH):
        descs = []
        for j in range(NGC):
            r = c * NGC + j
            descs.append(pltpu.async_copy(
                ps_hbm.at[si_v.at[r]], rs_v.at[pl.ds(j * GC, GC)], sem_s))
            descs.append(pltpu.async_copy(
                pd_hbm.at[di_v.at[r]], rd_v.at[pl.ds(j * GC, GC)], sem_d))
        for dsc in descs:
            dsc.wait()
        pltpu.sync_copy(rs_v, gs_hbm.at[pl.ds(base + c * CH, CH)])
        pltpu.sync_copy(rd_v, gd_hbm.at[pl.ds(base + c * CH, CH)])


def _sc_gather(psrc, pdst, si2, di2):
    f = pl.kernel(
        _gather_body,
        out_type=[
            jax.ShapeDtypeStruct((N_EDGES, DE), jnp.float32),
            jax.ShapeDtypeStruct((N_EDGES, DE), jnp.float32),
        ],
        mesh=plsc.VectorSubcoreMesh(core_axis_name="c", subcore_axis_name="s",
                                    num_cores=NC, num_subcores=NS),
        scratch_types=[
            pltpu.VMEM((IDX_ROWS, GC), jnp.int32),
            pltpu.VMEM((IDX_ROWS, GC), jnp.int32),
            pltpu.VMEM((CH, DE), jnp.float32),
            pltpu.VMEM((CH, DE), jnp.float32),
            pltpu.SemaphoreType.DMA,
            pltpu.SemaphoreType.DMA,
        ],
        compiler_params=pltpu.CompilerParams(use_tc_tiling_on_sc=False),
    )
    return f(psrc, pdst, si2, di2)


ROWS = N_EDGES * DE // 128
BE = 4000
NB = ROWS // BE


def _epi_body(gs, gd, ef, dd, w1e, w1d, w2, mavg, b1t, b2t, gt, bt, out):
    x = gs[...] + gd[...] + b1t[...]
    x = x + jnp.dot(ef[...], w1e[...], preferred_element_type=jnp.float32)
    x = x + jnp.dot(dd[...], w1d[...], preferred_element_type=jnp.float32)
    h1 = x * jax.nn.sigmoid(x)
    y = jnp.dot(h1, w2[...], preferred_element_type=jnp.float32) + b2t[...]
    h2 = y * jax.nn.sigmoid(y)
    r = ef[...] + h2
    m = jnp.dot(r, mavg[...], preferred_element_type=jnp.float32)
    q = r - m
    v = jnp.dot(q * q, mavg[...], preferred_element_type=jnp.float32)
    out[...] = q * lax.rsqrt(v + 1e-5) * gt[...] + bt[...]


def _epilogue(gs_p, gd_p, ef_p, d_p, bd1e, bd1d, bd2, mavg, b1t, b2t, gt, bt):
    big = pl.BlockSpec((BE, 128), lambda i: (i, 0))
    wsp = pl.BlockSpec((128, 128), lambda i: (0, 0))
    vsp = pl.BlockSpec((1, 128), lambda i: (0, 0))
    return pl.pallas_call(
        _epi_body,
        grid=(NB,),
        in_specs=[big, big, big, big, wsp, wsp, wsp, wsp, vsp, vsp, vsp, vsp],
        out_specs=big,
        out_shape=jax.ShapeDtypeStruct((ROWS, 128), jnp.float32),
    )(gs_p, gd_p, ef_p, d_p, bd1e, bd1d, bd2, mavg, b1t, b2t, gt, bt)


def kernel(node_scalars, edge_feats, d, src_idxs, dst_idxs,
           W1, b1, W2, b2, ln_g, ln_b):
    w_sd = jnp.concatenate([W1[0:D_NODE], W1[D_NODE:2 * D_NODE]], axis=1)
    eye8 = jnp.eye(8, dtype=jnp.float32)
    bd1e = jnp.kron(eye8, W1[2 * D_NODE:2 * D_NODE + DE])
    bd1d = jnp.kron(eye8, W1[2 * D_NODE + DE:2 * D_NODE + 2 * DE])
    bd2 = jnp.kron(eye8, W2)
    mavg = jnp.kron(eye8, jnp.full((DE, DE), 1.0 / DE, dtype=jnp.float32))
    b1t = jnp.tile(b1, 8)[None, :]
    b2t = jnp.tile(b2, 8)[None, :]
    gt = jnp.tile(ln_g, 8)[None, :]
    bt = jnp.tile(ln_b, 8)[None, :]

    psrc, pdst = _node_project(node_scalars, w_sd)

    si2 = src_idxs.astype(jnp.int32).reshape(NW, IDX_ROWS, GC)
    di2 = dst_idxs.astype(jnp.int32).reshape(NW, IDX_ROWS, GC)
    gs, gd = _sc_gather(psrc, pdst, si2, di2)

    out_p = _epilogue(
        gs.reshape(ROWS, 128), gd.reshape(ROWS, 128),
        edge_feats.reshape(ROWS, 128), d.reshape(ROWS, 128),
        bd1e, bd1d, bd2, mavg, b1t, b2t, gt, bt)
    return out_p.reshape(N_EDGES, DE)


# speedup vs baseline: 5.1409x; 1.2239x over previous
import jax
import jax.numpy as jnp
from jax import lax
from jax.experimental import pallas as pl
from jax.experimental.pallas import tpu as pltpu
from jax.experimental.pallas import tpu_sc as plsc

N_NODES = 10000
N_EDGES = 320000
D_NODE = 128
DE = 16

NC = 2
NS = 16
NW = NC * NS
GC = 80
CKE = 2560
GPC = CKE // GC
NCK = N_EDGES // CKE
CPW = -(-NCK // NW)


def _proj_body(ns_ref, w_ref, b_ref, ps_ref, pd_ref):
    p = jnp.dot(ns_ref[...], w_ref[...], preferred_element_type=jnp.float32)
    p = p + b_ref[...]
    ps_ref[...] = p[:, 0:DE]
    pd_ref[...] = p[:, DE:2 * DE]


def _node_project(node_scalars, w_sd, b12):
    return pl.pallas_call(
        _proj_body,
        out_shape=[
            jax.ShapeDtypeStruct((N_NODES, DE), jnp.float32),
            jax.ShapeDtypeStruct((N_NODES, DE), jnp.float32),
        ],
    )(node_scalars, w_sd, b12)


def _gather_body(ps_hbm, pd_hbm, si_hbm, di_hbm, gst_hbm, gdt_hbm,
                 si_v, di_v, rs_v, rd_v, sem_s, sem_d):
    wid = lax.axis_index("s") * NC + lax.axis_index("c")
    for k in range(CPW):
        ck = wid + k * NW

        @pl.when(ck < NCK)
        def _():
            pltpu.sync_copy(si_hbm.at[ck], si_v)
            pltpu.sync_copy(di_hbm.at[ck], di_v)
            descs = []
            for j in range(GPC):
                descs.append(pltpu.async_copy(
                    ps_hbm.at[si_v.at[j]], rs_v.at[pl.ds(j * GC, GC)], sem_s))
                descs.append(pltpu.async_copy(
                    pd_hbm.at[di_v.at[j]], rd_v.at[pl.ds(j * GC, GC)], sem_d))
            for dsc in descs:
                dsc.wait()
            row0 = ck * CKE
            pltpu.sync_copy(rs_v, gst_hbm.at[pl.ds(row0, CKE)])
            pltpu.sync_copy(rd_v, gdt_hbm.at[pl.ds(row0, CKE)])


def _sc_gather(psrc, pdst, si3, di3):
    f = pl.kernel(
        _gather_body,
        out_type=[
            jax.ShapeDtypeStruct((N_EDGES, DE), jnp.float32),
            jax.ShapeDtypeStruct((N_EDGES, DE), jnp.float32),
        ],
        mesh=plsc.VectorSubcoreMesh(core_axis_name="c", subcore_axis_name="s",
                                    num_cores=NC, num_subcores=NS),
        scratch_types=[
            pltpu.VMEM((GPC, GC), jnp.int32),
            pltpu.VMEM((GPC, GC), jnp.int32),
            pltpu.VMEM((CKE, DE), jnp.float32),
            pltpu.VMEM((CKE, DE), jnp.float32),
            pltpu.SemaphoreType.DMA,
            pltpu.SemaphoreType.DMA,
        ],
        compiler_params=pltpu.CompilerParams(use_tc_tiling_on_sc=False),
    )
    return f(psrc, pdst, si3, di3)


BL = 12800
NBL = N_EDGES // BL


def _epi_body(gs, gd, ef, dd, w1et, w1dt, w2t, b2c, gc, bc, out):
    x = gs[...] + gd[...]
    x = x + jnp.dot(w1et[...], ef[...], preferred_element_type=jnp.float32)
    x = x + jnp.dot(w1dt[...], dd[...], preferred_element_type=jnp.float32)
    h1 = x * jax.nn.sigmoid(x)
    y = jnp.dot(w2t[...], h1, preferred_element_type=jnp.float32)
    y = y + b2c[...][:, 0:1]
    h2 = y * jax.nn.sigmoid(y)
    r = ef[...] + h2
    m = jnp.mean(r, axis=0, keepdims=True)
    q = r - m
    v = jnp.mean(q * q, axis=0, keepdims=True)
    out[...] = q * lax.rsqrt(v + 1e-5) * gc[...][:, 0:1] + bc[...][:, 0:1]


def _epilogue(gst, gdt, eft, dt, w1et, w1dt, w2t, b2c, gc, bc):
    big = pl.BlockSpec((DE, BL), lambda i: (0, i))
    wsp = pl.BlockSpec((DE, DE), lambda i: (0, 0))
    vsp = pl.BlockSpec((DE, 128), lambda i: (0, 0))
    return pl.pallas_call(
        _epi_body,
        grid=(NBL,),
        in_specs=[big, big, big, big, wsp, wsp, wsp, vsp, vsp, vsp],
        out_specs=big,
        out_shape=jax.ShapeDtypeStruct((DE, N_EDGES), jnp.float32),
    )(gst, gdt, eft, dt, w1et, w1dt, w2t, b2c, gc, bc)


def kernel(node_scalars, edge_feats, d, src_idxs, dst_idxs,
           W1, b1, W2, b2, ln_g, ln_b):
    w_sd = jnp.concatenate([W1[0:D_NODE], W1[D_NODE:2 * D_NODE]], axis=1)
    b12 = 0.5 * jnp.concatenate([b1, b1])[None, :]
    w1et = W1[2 * D_NODE:2 * D_NODE + DE].T
    w1dt = W1[2 * D_NODE + DE:2 * D_NODE + 2 * DE].T
    w2t = W2.T
    b2c = jnp.tile(b2[:, None], (1, 128))
    gc = jnp.tile(ln_g[:, None], (1, 128))
    bc = jnp.tile(ln_b[:, None], (1, 128))

    psrc, pdst = _node_project(node_scalars, w_sd, b12)

    si3 = src_idxs.astype(jnp.int32).reshape(NCK, GPC, GC)
    di3 = dst_idxs.astype(jnp.int32).reshape(NCK, GPC, GC)
    gs, gd = _sc_gather(psrc, pdst, si3, di3)

    out_t = _epilogue(gs.T, gd.T, edge_feats.T, d.T,
                      w1et, w1dt, w2t, b2c, gc, bc)
    return out_t.T
